# baseline (speedup 1.0000x reference)
import jax
import jax.numpy as jnp
from jax.experimental import pallas as pl
from jax.experimental.pallas import tpu as pltpu

N_KEYS = 100000
BITS = 512
ROWS = 10000
NB = N_KEYS // ROWS


def _body(t_ref, w_ref, k_ref, o_ref):
    kb = k_ref[...].astype(jnp.bfloat16)
    d = jax.lax.dot_general(
        kb, w_ref[...], (((1,), (0,)), ((), ())),
        preferred_element_type=jnp.float32)
    t = t_ref[0]
    o_ref[...] = jnp.where(d[:, 0:1] <= t, 1.0, 0.0)


def kernel(query, keys):
    q = jnp.reshape(query, (BITS,))
    w = (1.0 - 2.0 * q).astype(jnp.bfloat16)
    wmat = jnp.tile(w[:, None], (1, 128))
    t = (1.0 - jnp.sum(q)).reshape(1)
    out = pl.pallas_call(
        _body,
        grid=(NB,),
        in_specs=[
            pl.BlockSpec(memory_space=pltpu.SMEM),
            pl.BlockSpec((BITS, 128), lambda i: (0, 0)),
            pl.BlockSpec((ROWS, BITS), lambda i: (i, 0)),
        ],
        out_specs=pl.BlockSpec((ROWS, 1), lambda i: (i, 0)),
        out_shape=jax.ShapeDtypeStruct((N_KEYS, 1), jnp.float32),
    )(t, wmat, keys)
    return out.reshape(N_KEYS)


# speedup vs baseline: 1.3718x; 1.3718x over previous
import jax
import jax.numpy as jnp
from jax.experimental import pallas as pl
from jax.experimental.pallas import tpu as pltpu

N_KEYS = 100000
BITS = 512
ROWS = 10000
NB = N_KEYS // ROWS


def _body(t_ref, w_ref, k_ref, o_ref):
    kb = k_ref[...].astype(jnp.bfloat16)
    d = jax.lax.dot_general(
        kb, w_ref[...], (((1,), (0,)), ((), ())),
        preferred_element_type=jnp.float32)
    t = t_ref[0]
    w = jnp.where(d[:, 0:1] <= t, 1.0, 0.0)
    o_ref[...] = w.reshape(1, 1, ROWS)


def kernel(query, keys):
    q = jnp.reshape(query, (BITS,))
    w = (1.0 - 2.0 * q).astype(jnp.bfloat16)
    wmat = jnp.tile(w[:, None], (1, 128))
    t = (1.0 - jnp.sum(q)).reshape(1)
    out = pl.pallas_call(
        _body,
        grid=(NB,),
        in_specs=[
            pl.BlockSpec(memory_space=pltpu.SMEM),
            pl.BlockSpec((BITS, 128), lambda i: (0, 0)),
            pl.BlockSpec((ROWS, BITS), lambda i: (i, 0)),
        ],
        out_specs=pl.BlockSpec((1, 1, ROWS), lambda i: (i, 0, 0)),
        out_shape=jax.ShapeDtypeStruct((NB, 1, ROWS), jnp.float32),
    )(t, wmat, keys)
    return out.reshape(N_KEYS)


# speedup vs baseline: 1.4163x; 1.0324x over previous
import jax
import jax.numpy as jnp
from jax.experimental import pallas as pl
from jax.experimental.pallas import tpu as pltpu

N_KEYS = 100000
BITS = 512
ROWS = 8192
NB = (N_KEYS + ROWS - 1) // ROWS
CH = ROWS // 128


def _body(t_ref, w_ref, k_ref, o_ref):
    kb = k_ref[...].astype(jnp.bfloat16)
    d = jax.lax.dot_general(
        kb, w_ref[...], (((1,), (0,)), ((), ())),
        preferred_element_type=jnp.float32)
    d3 = d.reshape(CH, 128, 128)
    row_i = jax.lax.broadcasted_iota(jnp.int32, (128, 128), 0)
    col_i = jax.lax.broadcasted_iota(jnp.int32, (128, 128), 1)
    eye = jnp.where(row_i == col_i, 1.0, 0.0)
    diag = jnp.sum(d3 * eye[None], axis=1)
    t = t_ref[0]
    o_ref[...] = jnp.where(diag <= t, 1.0, 0.0).reshape(1, CH, 128)


def kernel(query, keys):
    q = jnp.reshape(query, (BITS,))
    w = (1.0 - 2.0 * q).astype(jnp.bfloat16)
    wmat = jnp.tile(w[:, None], (1, 128))
    t = (1.0 - jnp.sum(q)).reshape(1)
    out = pl.pallas_call(
        _body,
        grid=(NB,),
        in_specs=[
            pl.BlockSpec(memory_space=pltpu.SMEM),
            pl.BlockSpec((BITS, 128), lambda i: (0, 0)),
            pl.BlockSpec((ROWS, BITS), lambda i: (i, 0)),
        ],
        out_specs=pl.BlockSpec((1, CH, 128), lambda i: (i, 0, 0)),
        out_shape=jax.ShapeDtypeStruct((NB, CH, 128), jnp.float32),
    )(t, wmat, keys)
    return out.reshape(NB * ROWS)[:N_KEYS]
